# baseline (speedup 1.0000x reference)
import functools

import jax
import jax.numpy as jnp
import numpy as np
from jax import lax
from jax.experimental import pallas as pl
from jax.experimental.pallas import tpu as pltpu
from jax.experimental.pallas import tpu_sc as plsc

_N = 1024


@functools.lru_cache(maxsize=None)
def _dft_basis(n):
    h = n // 2
    k = np.arange(h)
    j = np.arange(n)
    ang = (np.outer(k, j) % n).astype(np.float64) * (2.0 * np.pi / n)
    cosf = np.cos(ang).astype(np.float32)
    sinf = np.sin(ang).astype(np.float32)
    w = np.full((h,), 2.0)
    w[0] = 1.0
    cosi = (cosf.T * w).astype(np.float32)
    sini = (sinf.T * w).astype(np.float32)
    return cosf, sinf, cosi, sini


def _split_bf16(x):
    hi = x.astype(jnp.bfloat16)
    lo = (x - hi.astype(jnp.float32)).astype(jnp.bfloat16)
    return hi, lo


_MM = (((1,), (0,)), ((), ()))
_TT = (((1,), (1,)), ((), ()))

def _dot3(ah, al, bh, bl, dims):
    f = lambda u, v: lax.dot_general(u, v, dims,
                                     preferred_element_type=jnp.float32)
    return f(ah, bh) + (f(ah, bl) + f(al, bh))


def _phase_score_body(q_ref, cfh_ref, cfl_ref, sfh_ref, sfl_ref,
                      cih_ref, cil_ref, sih_ref, sil_ref,
                      ih_ref, il_ref, items_ref, idx_ref, part_ref,
                      isq_ref):
    x = q_ref[0]
    seq = x.shape[0]
    xh, xl = _split_bf16(x)
    a = _dot3(cfh_ref[...], cfl_ref[...], xh, xl, _MM)
    b = _dot3(sfh_ref[...], sfl_ref[...], xh, xl, _MM)
    mag2 = a * a + b * b
    inv = lax.rsqrt(mag2)
    good = mag2 > 0.0
    p = jnp.where(good, a * inv, 1.0)
    r = jnp.where(good, b * inv, 0.0)
    ph, pLo = _split_bf16(p)
    rh, rl = _split_bf16(r)
    u = (_dot3(cih_ref[...], cil_ref[...], ph, pLo, _MM)
         - _dot3(sih_ref[...], sil_ref[...], rh, rl, _MM))
    alt = (1 - 2 * (lax.broadcasted_iota(jnp.int32, (seq, 1), 0) % 2)
           ).astype(jnp.float32)
    fny = jnp.sum(x * alt, axis=0, keepdims=True)
    pny = jnp.where(fny != 0.0, jnp.sign(fny), 1.0)
    u = u + alt * pny
    uh, ul = _split_bf16(u)
    score = _dot3(uh, ul, ih_ref[...], il_ref[...], _TT)
    idx = jnp.argmax(score, axis=1).astype(jnp.int32)
    idx_ref[0, 0] = idx
    score_q = _dot3(xh, xl, ih_ref[...], il_ref[...], _TT)
    kiota = lax.broadcasted_iota(jnp.int32, score_q.shape, 1)
    val = jnp.sum(jnp.where(kiota == idx[:, None], score_q, 0.0), axis=1)
    q_sq = jnp.sum(x * x, axis=1)
    part_ref[0, 0] = q_sq - 2.0 * val
    it = items_ref[...]
    isq_ref[0, 0] = jnp.sum(it * it, axis=1)


def _sc_gather_add(item_sq, idx_flat, part_flat):
    info = plsc.get_sparse_core_info()
    nc, ns, nl = info.num_cores, info.num_subcores, info.num_lanes
    nw = nc * ns
    btot = idx_flat.shape[0]
    k_items = item_sq.shape[0]
    b_per_w = btot // nw
    mesh = plsc.VectorSubcoreMesh(core_axis_name="c", subcore_axis_name="s")

    chunk = 128

    @functools.partial(
        pl.kernel, mesh=mesh,
        out_type=jax.ShapeDtypeStruct((btot,), jnp.float32),
        scratch_types=[
            pltpu.VMEM((chunk,), jnp.int32),
            pltpu.VMEM((chunk,), jnp.float32),
            pltpu.VMEM((b_per_w,), jnp.float32),
            pltpu.VMEM((b_per_w,), jnp.float32),
            pltpu.SemaphoreType.DMA,
        ],
    )
    def gk(isq_hbm, idx_hbm, part_hbm, out_hbm, idx_v, g_v, part_v,
           loss_v, sem):
        wid = lax.axis_index("s") * nc + lax.axis_index("c")
        base = wid * b_per_w
        pltpu.sync_copy(part_hbm.at[pl.ds(base, b_per_w)], part_v)
        for c in range(b_per_w // chunk):
            pltpu.sync_copy(idx_hbm.at[pl.ds(base + c * chunk, chunk)],
                            idx_v)
            pltpu.async_copy(isq_hbm.at[idx_v], g_v, sem).wait()
            for j in range(chunk // nl):
                sl = pl.ds(c * chunk + j * nl, nl)
                loss_v[sl] = part_v[sl] + g_v[pl.ds(j * nl, nl)]
        pltpu.sync_copy(loss_v, out_hbm.at[pl.ds(base, b_per_w)])

    return gk(item_sq, idx_flat, part_flat)


def kernel(queries, items):
    bsz, seq, feat = queries.shape
    n_items = items.shape[0]
    half = seq // 2
    cf_np, sf_np, ci_np, si_np = _dft_basis(seq)

    cfh, cfl = _split_bf16(jnp.asarray(cf_np))
    sfh, sfl = _split_bf16(jnp.asarray(sf_np))
    cih, cil = _split_bf16(jnp.asarray(ci_np))
    sih, sil = _split_bf16(jnp.asarray(si_np))
    ih, il = _split_bf16(items)

    fwd_spec = pl.BlockSpec((half, seq), lambda b: (0, 0))
    inv_spec = pl.BlockSpec((seq, half), lambda b: (0, 0))
    it_spec = pl.BlockSpec((n_items, feat), lambda b: (0, 0))
    idx3, part3, isq3 = pl.pallas_call(
        _phase_score_body,
        grid=(bsz,),
        in_specs=[
            pl.BlockSpec((1, seq, feat), lambda b: (b, 0, 0)),
            fwd_spec, fwd_spec, fwd_spec, fwd_spec,
            inv_spec, inv_spec, inv_spec, inv_spec,
            it_spec, it_spec, it_spec,
        ],
        out_specs=[
            pl.BlockSpec((1, 1, seq), lambda b: (b, 0, 0)),
            pl.BlockSpec((1, 1, seq), lambda b: (b, 0, 0)),
            pl.BlockSpec((1, 1, n_items), lambda b: (0, 0, 0)),
        ],
        out_shape=[
            jax.ShapeDtypeStruct((bsz, 1, seq), jnp.int32),
            jax.ShapeDtypeStruct((bsz, 1, seq), jnp.float32),
            jax.ShapeDtypeStruct((1, 1, n_items), jnp.float32),
        ],
    )(queries, cfh, cfl, sfh, sfl, cih, cil, sih, sil, ih, il, items)

    loss = _sc_gather_add(isq3.reshape(-1), idx3.reshape(-1),
                          part3.reshape(-1))
    return loss.reshape(bsz, seq)


# speedup vs baseline: 1.1298x; 1.1298x over previous
import functools

import jax
import jax.numpy as jnp
import numpy as np
from jax import lax
from jax.experimental import pallas as pl
from jax.experimental.pallas import tpu as pltpu
from jax.experimental.pallas import tpu_sc as plsc

_N = 1024


@functools.lru_cache(maxsize=None)
def _dft_basis(n):
    h = n // 2
    k = np.arange(h)
    j = np.arange(n)
    ang = (np.outer(k, j) % n).astype(np.float64) * (2.0 * np.pi / n)
    cosf = np.cos(ang).astype(np.float32)
    sinf = np.sin(ang).astype(np.float32)
    w = np.full((h,), 2.0)
    w[0] = 1.0
    cosi = (cosf.T * w).astype(np.float32)
    sini = (sinf.T * w).astype(np.float32)
    return cosf, sinf, cosi, sini


def _split_bf16(x):
    hi = x.astype(jnp.bfloat16)
    lo = (x - hi.astype(jnp.float32)).astype(jnp.bfloat16)
    return hi, lo


_MM = (((1,), (0,)), ((), ()))
_TT = (((1,), (1,)), ((), ()))

def _dot3(ah, al, bh, bl, dims):
    f = lambda u, v: lax.dot_general(u, v, dims,
                                     preferred_element_type=jnp.float32)
    return f(ah, bh) + (f(ah, bl) + f(al, bh))


def _phase_score_body(q_ref, cfh_ref, cfl_ref, sfh_ref, sfl_ref,
                      cih_ref, cil_ref, sih_ref, sil_ref,
                      ih_ref, il_ref, idx_ref):
    x = q_ref[0]
    seq = x.shape[0]
    xh, xl = _split_bf16(x)
    a = _dot3(cfh_ref[...], cfl_ref[...], xh, xl, _MM)
    b = _dot3(sfh_ref[...], sfl_ref[...], xh, xl, _MM)
    mag2 = a * a + b * b
    inv = lax.rsqrt(mag2)
    good = mag2 > 0.0
    p = jnp.where(good, a * inv, 1.0)
    r = jnp.where(good, b * inv, 0.0)
    ph, pLo = _split_bf16(p)
    rh, rl = _split_bf16(r)
    u = (_dot3(cih_ref[...], cil_ref[...], ph, pLo, _MM)
         - _dot3(sih_ref[...], sil_ref[...], rh, rl, _MM))
    alt = (1 - 2 * (lax.broadcasted_iota(jnp.int32, (seq, 1), 0) % 2)
           ).astype(jnp.float32)
    fny = jnp.sum(x * alt, axis=0, keepdims=True)
    pny = jnp.where(fny != 0.0, jnp.sign(fny), 1.0)
    u = u + alt * pny
    uh, ul = _split_bf16(u)
    score = _dot3(uh, ul, ih_ref[...], il_ref[...], _TT)
    idx_ref[...] = jnp.argmax(score, axis=1).astype(jnp.int32)


def _loss_body(q_ref, g_ref, loss_ref):
    hf = g_ref.shape[1]
    g32 = g_ref[...]
    g_lo = lax.bitcast_convert_type(g32 << 16, jnp.float32)
    g_hi = lax.bitcast_convert_type(g32 & jnp.int32(-65536), jnp.float32)
    x = q_ref[0]
    d_lo = x[:, :hf] - g_lo
    d_hi = x[:, hf:] - g_hi
    loss_ref[...] = jnp.sum(d_lo * d_lo + d_hi * d_hi, axis=1)


def _sc_gather(items_pk, idx_flat):
    info = plsc.get_sparse_core_info()
    nc, ns = info.num_cores, info.num_subcores
    nw = nc * ns
    btot = idx_flat.shape[0]
    d_pk = items_pk.shape[1]
    b_per_w = btot // nw
    chunk = 128
    mesh = plsc.VectorSubcoreMesh(core_axis_name="c", subcore_axis_name="s")

    @functools.partial(
        pl.kernel, mesh=mesh,
        out_type=jax.ShapeDtypeStruct((btot, d_pk), jnp.int32),
        scratch_types=[
            pltpu.VMEM((chunk,), jnp.int32),
            pltpu.VMEM((chunk, d_pk), jnp.int32),
            pltpu.SemaphoreType.DMA,
        ],
    )
    def gk(items_hbm, idx_hbm, out_hbm, idx_v, rows_v, sem):
        wid = lax.axis_index("s") * nc + lax.axis_index("c")
        base = wid * b_per_w
        for c in range(b_per_w // chunk):
            off = base + c * chunk
            pltpu.sync_copy(idx_hbm.at[pl.ds(off, chunk)], idx_v)
            pltpu.async_copy(items_hbm.at[idx_v], rows_v, sem).wait()
            pltpu.sync_copy(rows_v, out_hbm.at[pl.ds(off, chunk)])

    return gk(items_pk, idx_flat)


def kernel(queries, items):
    bsz, seq, feat = queries.shape
    n_items = items.shape[0]
    half = seq // 2
    cf_np, sf_np, ci_np, si_np = _dft_basis(seq)

    cfh, cfl = _split_bf16(jnp.asarray(cf_np))
    sfh, sfl = _split_bf16(jnp.asarray(sf_np))
    cih, cil = _split_bf16(jnp.asarray(ci_np))
    sih, sil = _split_bf16(jnp.asarray(si_np))
    ih, il = _split_bf16(items)

    fwd_spec = pl.BlockSpec((half, seq), lambda b: (0, 0))
    inv_spec = pl.BlockSpec((seq, half), lambda b: (0, 0))
    it_spec = pl.BlockSpec((n_items, feat), lambda b: (0, 0))
    idx_flat = pl.pallas_call(
        _phase_score_body,
        grid=(bsz,),
        in_specs=[
            pl.BlockSpec((1, seq, feat), lambda b: (b, 0, 0)),
            fwd_spec, fwd_spec, fwd_spec, fwd_spec,
            inv_spec, inv_spec, inv_spec, inv_spec,
            it_spec, it_spec,
        ],
        out_specs=pl.BlockSpec((seq,), lambda b: (b,)),
        out_shape=jax.ShapeDtypeStruct((bsz * seq,), jnp.int32),
    )(queries, cfh, cfl, sfh, sfl, cih, cil, sih, sil, ih, il)

    items_pk = lax.bitcast_convert_type(
        jnp.stack([ih[:, :feat // 2], ih[:, feat // 2:]], axis=-1),
        jnp.int32)
    gathered = _sc_gather(items_pk, idx_flat)

    loss = pl.pallas_call(
        _loss_body,
        grid=(bsz,),
        in_specs=[
            pl.BlockSpec((1, seq, feat), lambda b: (b, 0, 0)),
            pl.BlockSpec((seq, feat // 2), lambda b: (b, 0)),
        ],
        out_specs=pl.BlockSpec((seq,), lambda b: (b,)),
        out_shape=jax.ShapeDtypeStruct((bsz * seq,), jnp.float32),
    )(queries, gathered)

    return loss.reshape(bsz, seq)
